# baseline (speedup 1.0000x reference)
import jax
import jax.numpy as jnp
from jax.experimental import pallas as pl

A = 4
NEIGHBOR_NUM = 32


def _knn(vertices, k):
    inner = jnp.einsum('bid,bjd->bij', vertices, vertices)
    sq = jnp.sum(vertices * vertices, axis=-1)
    dist = sq[:, :, None] + sq[:, None, :] - 2.0 * inner
    _, idx = jax.lax.top_k(-dist, k + 1)
    return idx[:, :, 1:]


def _gather(x, idx):
    return jax.vmap(lambda xb, ib: xb[ib])(x, idx)


def _conv_surface(idx, vertices, W0, b0):
    nbr = _gather(vertices, idx)
    support = nbr - vertices[:, :, None, :]
    support = support / (jnp.linalg.norm(support, axis=-1, keepdims=True) + 1e-8)
    theta = jnp.einsum('bvkd,dao->bvkao', support, W0) + b0[None, None, None, :, :]
    theta = jax.nn.relu(theta)
    fm = jnp.max(theta, axis=2)
    return jnp.transpose(fm, (0, 3, 1, 2))


def _equi_conv(idx, vertices, fm, Wc, Wn, Wd, b):
    center = jnp.einsum('bcva,co->bova', fm, Wc)
    fm_t = jnp.transpose(fm, (0, 2, 1, 3))
    nbr_f = _gather(fm_t, idx)
    nbr_term = jnp.einsum('bvkca,co->bvkoa', nbr_f, Wn)
    nbr_v = _gather(vertices, idx)
    support = nbr_v - vertices[:, :, None, :]
    support = support / (jnp.linalg.norm(support, axis=-1, keepdims=True) + 1e-8)
    dir_term = jnp.einsum('bvkd,do->bvko', support, Wd)[..., None]
    agg = jnp.max(nbr_term + dir_term, axis=2)
    return center + jnp.transpose(agg, (0, 2, 1, 3)) + b[None, :, None, None]


def _batchnorm(x, gamma, beta, eps=1e-5):
    mean = jnp.mean(x, axis=(0, 2, 3), keepdims=True)
    var = jnp.var(x, axis=(0, 2, 3), keepdims=True)
    return gamma[None, :, None, None] * (x - mean) / jnp.sqrt(var + eps) + beta[None, :, None, None]


def _pool_layer(vertices, fm, rate, nbr_num):
    bs, v, _ = vertices.shape
    pool_num = v // rate
    idx = _knn(vertices, nbr_num)
    fm_t = jnp.transpose(fm, (0, 2, 1, 3))
    nbr_f = _gather(fm_t, idx)
    fm_max = jnp.maximum(fm_t, jnp.max(nbr_f, axis=2))
    sel = jnp.arange(pool_num) * rate
    v_pool = vertices[:, sel, :]
    fm_pool = jnp.transpose(fm_max[:, sel, :, :], (0, 2, 1, 3))
    return v_pool, fm_pool


def _anchor_pool(fm):
    return jnp.max(fm, axis=3)


def _identity_body(x_ref, o_ref):
    o_ref[...] = x_ref[...]


def _pallas_identity(x):
    return pl.pallas_call(
        _identity_body,
        out_shape=jax.ShapeDtypeStruct(x.shape, x.dtype),
    )(x)


def kernel(vertices, W0, b0, Wc1, Wn1, Wd1, b1, Wc2, Wn2, Wd2, b2, Wc3, Wn3, Wd3, b3, Wc4, Wn4, Wd4, b4, g1, be1, g2, be2, g3, be3):
    vertices = _pallas_identity(vertices)
    idx = _knn(vertices, NEIGHBOR_NUM)
    fm_0 = jax.nn.relu(_conv_surface(idx, vertices, W0, b0))
    fm_1 = jax.nn.relu(_batchnorm(_equi_conv(idx, vertices, fm_0, Wc1, Wn1, Wd1, b1), g1, be1))
    v1, fmp1 = _pool_layer(vertices, fm_1, 4, 4)
    idx = _knn(v1, min(NEIGHBOR_NUM, v1.shape[1] // 8))
    fm_2 = jax.nn.relu(_batchnorm(_equi_conv(idx, v1, fmp1, Wc2, Wn2, Wd2, b2), g2, be2))
    fm_3 = jax.nn.relu(_batchnorm(_equi_conv(idx, v1, fm_2, Wc3, Wn3, Wd3, b3), g3, be3))
    v2, fmp2 = _pool_layer(v1, fm_3, 4, 4)
    idx = _knn(v2, min(NEIGHBOR_NUM, v2.shape[1] // 8))
    fm_4 = _equi_conv(idx, v2, fmp2, Wc4, Wn4, Wd4, b4)
    fea0 = jnp.transpose(_anchor_pool(fm_0), (0, 2, 1))
    fea1 = jnp.transpose(_anchor_pool(fm_1), (0, 2, 1))
    fea2 = jnp.transpose(_anchor_pool(fm_2), (0, 2, 1))
    fea3 = jnp.transpose(_anchor_pool(fm_3), (0, 2, 1))
    fea4 = jnp.transpose(_anchor_pool(fm_4), (0, 2, 1))
    return (vertices, fea0, fea1, v1, fea2, fea3, v2, fea4)


# speedup vs baseline: 1.3600x; 1.3600x over previous
import jax
import jax.numpy as jnp
from jax.experimental import pallas as pl

A = 4
K = 32
RB = 128


def _knn_body(verts_ref, va_ref, vaT_ref, idx_ref, sx_ref, sy_ref, sz_ref):
    i = pl.program_id(1)
    vr = verts_ref[0]
    va = va_ref[0]
    vaT = vaT_ref[0]
    V = va.shape[0]

    sq_r = jnp.sum(vr * vr, axis=1, keepdims=True)
    sq_a = jnp.sum(vaT * vaT, axis=0, keepdims=True)
    mm = jnp.dot(vr.astype(jnp.bfloat16), vaT.astype(jnp.bfloat16),
                 preferred_element_type=jnp.float32)
    dist = sq_r + sq_a - 2.0 * mm

    col = jax.lax.broadcasted_iota(jnp.int32, (RB, V), 1)

    lane32 = jax.lax.broadcasted_iota(jnp.int32, (RB, K), 1)
    inf = jnp.float32(jnp.inf)

    def body(t, carry):
        dist, idxa, sxa, sya, sza = carry
        m = jnp.min(dist, axis=1, keepdims=True)
        idx_t = jnp.min(jnp.where(dist <= m, col, V), axis=1, keepdims=True)
        onehot = col == idx_t
        nbr = jnp.dot(onehot.astype(jnp.float32), va,
                      preferred_element_type=jnp.float32,
                      precision=jax.lax.Precision.HIGHEST)
        d3 = nbr - vr
        n2 = jnp.sum(d3 * d3, axis=1, keepdims=True)
        inv = 1.0 / (jnp.sqrt(n2) + 1e-8)
        s3 = d3 * inv
        sel = lane32 == t - 1
        idxa = jnp.where(sel, idx_t, idxa)
        sxa = jnp.where(sel, s3[:, 0:1], sxa)
        sya = jnp.where(sel, s3[:, 1:2], sya)
        sza = jnp.where(sel, s3[:, 2:3], sza)
        dist = jnp.where(onehot, inf, dist)
        return dist, idxa, sxa, sya, sza

    z = jnp.zeros((RB, K), jnp.float32)
    iz = jnp.zeros((RB, K), jnp.int32)
    _, idxa, sxa, sya, sza = jax.lax.fori_loop(0, K + 1, body, (dist, iz, z, z, z))
    idx_ref[0] = idxa
    sx_ref[0] = sxa
    sy_ref[0] = sya
    sz_ref[0] = sza


def _knn_pallas(vertices):
    B, V, _ = vertices.shape
    vaT = jnp.transpose(vertices, (0, 2, 1))
    f32 = jnp.float32
    out_shapes = (
        jax.ShapeDtypeStruct((B, V, K), jnp.int32),
        jax.ShapeDtypeStruct((B, V, K), f32),
        jax.ShapeDtypeStruct((B, V, K), f32),
        jax.ShapeDtypeStruct((B, V, K), f32),
    )
    blk_out = pl.BlockSpec((1, RB, K), lambda b, i: (b, i, 0))
    return pl.pallas_call(
        _knn_body,
        grid=(B, V // RB),
        in_specs=[
            pl.BlockSpec((1, RB, 3), lambda b, i: (b, i, 0)),
            pl.BlockSpec((1, V, 3), lambda b, i: (b, 0, 0)),
            pl.BlockSpec((1, 3, V), lambda b, i: (b, 0, 0)),
        ],
        out_specs=(blk_out, blk_out, blk_out, blk_out),
        out_shape=out_shapes,
    )(vertices, vertices, vaT)


def _gather(x, idx):
    return jax.vmap(lambda xb, ib: xb[ib])(x, idx)


def _conv_surface(support, W0, b0):
    theta = jnp.einsum('bvkd,dao->bvkao', support, W0) + b0[None, None, None, :, :]
    theta = jax.nn.relu(theta)
    fm = jnp.max(theta, axis=2)
    return jnp.transpose(fm, (0, 3, 1, 2))


def _equi_conv(idx, support, fm, Wc, Wn, Wd, b):
    center = jnp.einsum('bcva,co->bova', fm, Wc)
    fm_t = jnp.transpose(fm, (0, 2, 1, 3))
    nbr_f = _gather(fm_t, idx)
    nbr_term = jnp.einsum('bvkca,co->bvkoa', nbr_f, Wn)
    dir_term = jnp.einsum('bvkd,do->bvko', support, Wd)[..., None]
    agg = jnp.max(nbr_term + dir_term, axis=2)
    return center + jnp.transpose(agg, (0, 2, 1, 3)) + b[None, :, None, None]


def _batchnorm(x, gamma, beta, eps=1e-5):
    mean = jnp.mean(x, axis=(0, 2, 3), keepdims=True)
    var = jnp.var(x, axis=(0, 2, 3), keepdims=True)
    return gamma[None, :, None, None] * (x - mean) / jnp.sqrt(var + eps) + beta[None, :, None, None]


def _pool_layer(vertices, fm, idx4, rate):
    bs, v, _ = vertices.shape
    pool_num = v // rate
    fm_t = jnp.transpose(fm, (0, 2, 1, 3))
    nbr_f = _gather(fm_t, idx4)
    fm_max = jnp.maximum(fm_t, jnp.max(nbr_f, axis=2))
    sel = jnp.arange(pool_num) * rate
    v_pool = vertices[:, sel, :]
    fm_pool = jnp.transpose(fm_max[:, sel, :, :], (0, 2, 1, 3))
    return v_pool, fm_pool


def _anchor_pool(fm):
    return jnp.max(fm, axis=3)


def kernel(vertices, W0, b0, Wc1, Wn1, Wd1, b1, Wc2, Wn2, Wd2, b2, Wc3, Wn3, Wd3, b3, Wc4, Wn4, Wd4, b4, g1, be1, g2, be2, g3, be3):
    idx0, sx0, sy0, sz0 = _knn_pallas(vertices)
    sup0 = jnp.stack([sx0, sy0, sz0], axis=-1)
    fm_0 = jax.nn.relu(_conv_surface(sup0, W0, b0))
    fm_1 = jax.nn.relu(_batchnorm(_equi_conv(idx0, sup0, fm_0, Wc1, Wn1, Wd1, b1), g1, be1))
    v1, fmp1 = _pool_layer(vertices, fm_1, idx0[:, :, :4], 4)

    idx1, sx1, sy1, sz1 = _knn_pallas(v1)
    sup1 = jnp.stack([sx1, sy1, sz1], axis=-1)
    fm_2 = jax.nn.relu(_batchnorm(_equi_conv(idx1, sup1, fmp1, Wc2, Wn2, Wd2, b2), g2, be2))
    fm_3 = jax.nn.relu(_batchnorm(_equi_conv(idx1, sup1, fm_2, Wc3, Wn3, Wd3, b3), g3, be3))
    v2, fmp2 = _pool_layer(v1, fm_3, idx1[:, :, :4], 4)

    idx2, sx2, sy2, sz2 = _knn_pallas(v2)
    sup2 = jnp.stack([sx2, sy2, sz2], axis=-1)
    fm_4 = _equi_conv(idx2, sup2, fmp2, Wc4, Wn4, Wd4, b4)

    fea0 = jnp.transpose(_anchor_pool(fm_0), (0, 2, 1))
    fea1 = jnp.transpose(_anchor_pool(fm_1), (0, 2, 1))
    fea2 = jnp.transpose(_anchor_pool(fm_2), (0, 2, 1))
    fea3 = jnp.transpose(_anchor_pool(fm_3), (0, 2, 1))
    fea4 = jnp.transpose(_anchor_pool(fm_4), (0, 2, 1))
    return (vertices, fea0, fea1, v1, fea2, fea3, v2, fea4)


# speedup vs baseline: 1.9713x; 1.4495x over previous
import functools

import jax
import jax.numpy as jnp
from jax import lax
from jax.experimental import pallas as pl
from jax.experimental.pallas import tpu as pltpu
from jax.experimental.pallas import tpu_sc as plsc

A = 4
K = 32
RB = 128
NW = 32

f32 = jnp.float32
bf16 = jnp.bfloat16


def _r16(x):
    return x.astype(bf16).astype(f32)


def _knn_body(verts_ref, va_ref, vaT_ref, idx_ref, sx_ref, sy_ref, sz_ref):
    vr = verts_ref[0]
    va = va_ref[0]
    vaT = vaT_ref[0]
    V = va.shape[0]

    sq_r = jnp.sum(vr * vr, axis=1, keepdims=True)
    sq_a = jnp.sum(vaT * vaT, axis=0, keepdims=True)
    mm = jnp.dot(vr.astype(bf16), vaT.astype(bf16),
                 preferred_element_type=f32)
    dist = sq_r + sq_a - 2.0 * mm

    col = lax.broadcasted_iota(jnp.int32, (RB, V), 1)
    lane32 = lax.broadcasted_iota(jnp.int32, (RB, K), 1)
    inf = jnp.float32(jnp.inf)

    def body(t, carry):
        dist, idxa, sxa, sya, sza = carry
        m = jnp.min(dist, axis=1, keepdims=True)
        idx_t = jnp.min(jnp.where(dist <= m, col, V), axis=1, keepdims=True)
        onehot = col == idx_t
        nbr = jnp.dot(onehot.astype(f32), va, preferred_element_type=f32,
                      precision=lax.Precision.HIGHEST)
        d3 = nbr - vr
        n2 = jnp.sum(d3 * d3, axis=1, keepdims=True)
        inv = 1.0 / (jnp.sqrt(n2) + 1e-8)
        s3 = d3 * inv
        sel = lane32 == t - 1
        idxa = jnp.where(sel, idx_t, idxa)
        sxa = jnp.where(sel, s3[:, 0:1], sxa)
        sya = jnp.where(sel, s3[:, 1:2], sya)
        sza = jnp.where(sel, s3[:, 2:3], sza)
        dist = jnp.where(onehot, inf, dist)
        return dist, idxa, sxa, sya, sza

    z = jnp.zeros((RB, K), f32)
    iz = jnp.zeros((RB, K), jnp.int32)
    _, idxa, sxa, sya, sza = lax.fori_loop(0, K + 1, body, (dist, iz, z, z, z))
    idx_ref[0] = idxa
    sx_ref[0] = sxa
    sy_ref[0] = sya
    sz_ref[0] = sza


def _knn_pallas(vertices):
    B, V, _ = vertices.shape
    vaT = jnp.transpose(vertices, (0, 2, 1))
    out_shapes = (
        jax.ShapeDtypeStruct((B, V, K), jnp.int32),
        jax.ShapeDtypeStruct((B, V, K), f32),
        jax.ShapeDtypeStruct((B, V, K), f32),
        jax.ShapeDtypeStruct((B, V, K), f32),
    )
    blk_out = pl.BlockSpec((1, RB, K), lambda b, i: (b, i, 0))
    return pl.pallas_call(
        _knn_body,
        grid=(B, V // RB),
        in_specs=[
            pl.BlockSpec((1, RB, 3), lambda b, i: (b, i, 0)),
            pl.BlockSpec((1, V, 3), lambda b, i: (b, 0, 0)),
            pl.BlockSpec((1, 3, V), lambda b, i: (b, 0, 0)),
        ],
        out_specs=(blk_out, blk_out, blk_out, blk_out),
        out_shape=out_shapes,
    )(vertices, vertices, vaT)


RBS = 128


def _surface_body(sx_ref, sy_ref, sz_ref, w_ref, b_ref, fm_ref, fea_ref):
    w = w_ref[...]
    AO = w.shape[1]
    sx = sx_ref[...]
    sy = sy_ref[...]
    sz = sz_ref[...]
    sup3 = jnp.concatenate([sx, sy, sz], axis=1)
    theta = jnp.dot(sup3.astype(bf16), w.astype(bf16),
                    preferred_element_type=f32) + b_ref[...]
    theta = jnp.maximum(theta, 0.0)
    theta3 = theta.reshape(RBS, K, AO)
    fm = jnp.max(theta3, axis=1)
    fm_ref[...] = fm
    O = AO // A
    fea = fm[:, 0:O]
    for a in range(1, A):
        fea = jnp.maximum(fea, fm[:, a * O:(a + 1) * O])
    fea_ref[...] = fea


def _surface_pallas(sxf, syf, szf, W0r, b0r, N):
    AO = W0r.shape[1]
    O = AO // A
    return pl.pallas_call(
        _surface_body,
        grid=(N // RBS,),
        in_specs=[
            pl.BlockSpec((RBS * K, 1), lambda i: (i, 0)),
            pl.BlockSpec((RBS * K, 1), lambda i: (i, 0)),
            pl.BlockSpec((RBS * K, 1), lambda i: (i, 0)),
            pl.BlockSpec((3, AO), lambda i: (0, 0)),
            pl.BlockSpec((1, AO), lambda i: (0, 0)),
        ],
        out_specs=(
            pl.BlockSpec((RBS, AO), lambda i: (i, 0)),
            pl.BlockSpec((RBS, O), lambda i: (i, 0)),
        ),
        out_shape=(
            jax.ShapeDtypeStruct((N, AO), f32),
            jax.ShapeDtypeStruct((N, O), f32),
        ),
    )(sxf, syf, szf, W0r, b0r)


def _prep_body(f_ref, wc_ref, wn_ref, c_ref, p_ref):
    fm = f_ref[...].astype(bf16)
    AC = fm.shape[1]
    C = AC // A
    wc = wc_ref[...].astype(bf16)
    wn = wn_ref[...].astype(bf16)
    cs = [jnp.dot(fm[:, a * C:(a + 1) * C], wc, preferred_element_type=f32)
          for a in range(A)]
    ps = [jnp.dot(fm[:, a * C:(a + 1) * C], wn, preferred_element_type=f32)
          for a in range(A)]
    c_ref[...] = jnp.concatenate(cs, axis=1)
    p_ref[...] = jnp.concatenate(ps, axis=1)


def _prep_pallas(F, Wc, Wn, rbp):
    N, AC = F.shape
    AO = A * Wc.shape[1]
    return pl.pallas_call(
        _prep_body,
        grid=(N // rbp,),
        in_specs=[
            pl.BlockSpec((rbp, AC), lambda i: (i, 0)),
            pl.BlockSpec(Wc.shape, lambda i: (0, 0)),
            pl.BlockSpec(Wn.shape, lambda i: (0, 0)),
        ],
        out_specs=(
            pl.BlockSpec((rbp, AO), lambda i: (i, 0)),
            pl.BlockSpec((rbp, AO), lambda i: (i, 0)),
        ),
        out_shape=(
            jax.ShapeDtypeStruct((N, AO), f32),
            jax.ShapeDtypeStruct((N, AO), f32),
        ),
    )(F, Wc, Wn)


def _make_sc_agg(N, KN, AO, O, CR, with_dir):
    nv = N // NW
    nch = nv // CR
    mesh = plsc.VectorSubcoreMesh(core_axis_name="c", subcore_axis_name="s")
    scratch = [
        pltpu.VMEM((CR * KN,), jnp.int32),
        pltpu.VMEM((CR * KN, AO), f32),
        pltpu.VMEM((CR, AO), f32),
        pltpu.SemaphoreType.DMA,
    ]
    if with_dir:
        scratch += [pltpu.VMEM((CR * KN, O), f32)]

    def body(*refs):
        if with_dir:
            (proj, idxg, dirh, out, idx_v, rows_v, out_v, sem, dir_v) = refs
        else:
            (proj, idxg, out, idx_v, rows_v, out_v, sem) = refs
        wid = lax.axis_index("s") * 2 + lax.axis_index("c")
        base = wid * nv

        def chunk(ci, carry):
            v0 = base + ci * CR
            pltpu.sync_copy(idxg.at[pl.ds(v0 * KN, CR * KN)], idx_v)
            if with_dir:
                pltpu.sync_copy(dirh.at[pl.ds(v0 * KN, CR * KN)], dir_v)
            pltpu.async_copy(proj.at[idx_v], rows_v, sem).wait()
            for r in range(CR):
                def kbody(k, accs):
                    rk = r * KN + k
                    if with_dir:
                        dirs = [dir_v[rk, pl.ds(jo * 16, 16)]
                                for jo in range(O // 16)]
                    new = []
                    for j in range(AO // 16):
                        v = rows_v[rk, pl.ds(j * 16, 16)]
                        if with_dir:
                            v = v + dirs[j % (O // 16)]
                        new.append(jnp.maximum(accs[j], v))
                    return tuple(new)

                accs = lax.fori_loop(
                    0, KN, kbody,
                    tuple(jnp.full((16,), -jnp.inf, f32)
                          for _ in range(AO // 16)))
                for j in range(AO // 16):
                    out_v[r, pl.ds(j * 16, 16)] = accs[j]
            pltpu.sync_copy(out_v, out.at[pl.ds(v0, CR)])
            return carry

        lax.fori_loop(0, nch, chunk, 0)

    return functools.partial(
        pl.kernel, body,
        out_type=jax.ShapeDtypeStruct((N, AO), f32),
        mesh=mesh,
        scratch_types=scratch,
    )()


def _sc_agg(proj, idxg, dirf, CR):
    N, AO = proj.shape
    KN = idxg.shape[0] // N
    O = dirf.shape[1]
    fn = _make_sc_agg(N, KN, AO, O, CR, True)
    return fn(proj, idxg, dirf)


def _sc_pool(F, idxp, CR):
    N = idxp.shape[0] // 8
    AO = F.shape[1]
    fn = _make_sc_agg(N, 8, AO, AO, CR, False)
    return fn(F, idxp)


RBD = 128


def _dir_body(sx_ref, sy_ref, sz_ref, wd_ref, dir_ref):
    wd = wd_ref[...]
    sup3 = jnp.concatenate([sx_ref[...], sy_ref[...], sz_ref[...]], axis=1)
    dir_ref[...] = jnp.dot(sup3.astype(bf16), wd.astype(bf16),
                           preferred_element_type=f32)


def _dir_pallas(sxf, syf, szf, Wdr):
    NK = sxf.shape[0]
    O = Wdr.shape[1]
    return pl.pallas_call(
        _dir_body,
        grid=(NK // (RBD * K),),
        in_specs=[
            pl.BlockSpec((RBD * K, 1), lambda i: (i, 0)),
            pl.BlockSpec((RBD * K, 1), lambda i: (i, 0)),
            pl.BlockSpec((RBD * K, 1), lambda i: (i, 0)),
            pl.BlockSpec((3, O), lambda i: (0, 0)),
        ],
        out_specs=pl.BlockSpec((RBD * K, O), lambda i: (i, 0)),
        out_shape=jax.ShapeDtypeStruct((NK, O), f32),
    )(sxf, syf, szf, Wdr)


RBE = 512


def _e1_body(c_ref, a_ref, b_ref, pre_ref, ps_ref, pq_ref):
    pre = c_ref[...] + a_ref[...] + b_ref[...]
    pre_ref[...] = pre
    ps_ref[0] = jnp.sum(pre, axis=0, keepdims=True)
    pq_ref[0] = jnp.sum(pre * pre, axis=0, keepdims=True)


def _e1_pallas(center, agg, btile):
    N, AO = center.shape
    nblk = N // RBE
    return pl.pallas_call(
        _e1_body,
        grid=(nblk,),
        in_specs=[
            pl.BlockSpec((RBE, AO), lambda i: (i, 0)),
            pl.BlockSpec((RBE, AO), lambda i: (i, 0)),
            pl.BlockSpec((1, AO), lambda i: (0, 0)),
        ],
        out_specs=(
            pl.BlockSpec((RBE, AO), lambda i: (i, 0)),
            pl.BlockSpec((1, 1, AO), lambda i: (i, 0, 0)),
            pl.BlockSpec((1, 1, AO), lambda i: (i, 0, 0)),
        ),
        out_shape=(
            jax.ShapeDtypeStruct((N, AO), f32),
            jax.ShapeDtypeStruct((nblk, 1, AO), f32),
            jax.ShapeDtypeStruct((nblk, 1, AO), f32),
        ),
    )(center, agg, btile)


def _e2_body(pre_ref, ps_ref, pq_ref, g_ref, be_ref, cnt_ref, f_ref, fea_ref):
    AO = pre_ref.shape[1]
    O = g_ref.shape[1]
    s_ao = jnp.sum(ps_ref[...].reshape(-1, AO), axis=0, keepdims=True)
    q_ao = jnp.sum(pq_ref[...].reshape(-1, AO), axis=0, keepdims=True)
    s_o = s_ao[:, 0:O]
    q_o = q_ao[:, 0:O]
    for a in range(1, A):
        s_o = s_o + s_ao[:, a * O:(a + 1) * O]
        q_o = q_o + q_ao[:, a * O:(a + 1) * O]
    cnt = cnt_ref[0, 0]
    mean = s_o / cnt
    var = q_o / cnt - mean * mean
    scale = g_ref[...] / jnp.sqrt(var + 1e-5)
    shift = be_ref[...] - mean * scale
    scale_ao = jnp.concatenate([scale] * A, axis=1)
    shift_ao = jnp.concatenate([shift] * A, axis=1)
    y = jnp.maximum(pre_ref[...] * scale_ao + shift_ao, 0.0)
    f_ref[...] = y
    fea = y[:, 0:O]
    for a in range(1, A):
        fea = jnp.maximum(fea, y[:, a * O:(a + 1) * O])
    fea_ref[...] = fea


def _e2_pallas(pre, ps, pq, g, be):
    N, AO = pre.shape
    nblk = ps.shape[0]
    O = AO // A
    cnt = jnp.full((1, 1), float(N * A), f32)
    return pl.pallas_call(
        _e2_body,
        grid=(N // RBE,),
        in_specs=[
            pl.BlockSpec((RBE, AO), lambda i: (i, 0)),
            pl.BlockSpec((nblk, 1, AO), lambda i: (0, 0, 0)),
            pl.BlockSpec((nblk, 1, AO), lambda i: (0, 0, 0)),
            pl.BlockSpec((1, O), lambda i: (0, 0)),
            pl.BlockSpec((1, O), lambda i: (0, 0)),
            pl.BlockSpec((1, 1), lambda i: (0, 0)),
        ],
        out_specs=(
            pl.BlockSpec((RBE, AO), lambda i: (i, 0)),
            pl.BlockSpec((RBE, O), lambda i: (i, 0)),
        ),
        out_shape=(
            jax.ShapeDtypeStruct((N, AO), f32),
            jax.ShapeDtypeStruct((N, O), f32),
        ),
    )(pre, ps, pq, g, be, cnt)


def _e4_body(c_ref, a_ref, b_ref, fea_ref):
    AO = c_ref.shape[1]
    O = fea_ref.shape[1]
    y = c_ref[...] + a_ref[...] + b_ref[...]
    fea = y[:, 0:O]
    for a in range(1, A):
        fea = jnp.maximum(fea, y[:, a * O:(a + 1) * O])
    fea_ref[...] = fea


def _e4_pallas(center, agg, btile, rbe):
    N, AO = center.shape
    O = AO // A
    return pl.pallas_call(
        _e4_body,
        grid=(N // rbe,),
        in_specs=[
            pl.BlockSpec((rbe, AO), lambda i: (i, 0)),
            pl.BlockSpec((rbe, AO), lambda i: (i, 0)),
            pl.BlockSpec((1, AO), lambda i: (0, 0)),
        ],
        out_specs=pl.BlockSpec((rbe, O), lambda i: (i, 0)),
        out_shape=jax.ShapeDtypeStruct((N, O), f32),
    )(center, agg, btile)


def _bd(W):
    eye = jnp.eye(A, dtype=f32)
    return jnp.kron(eye, W)


def _flat_sup(s):
    return s.reshape(-1)


def _equi_layer(F, idxg, sxf, syf, szf, Wc, Wn, Wd, b, rbp, cr):
    center, proj = _prep_pallas(F, Wc, Wn, rbp)
    dirf = _dir_pallas(sxf[:, None], syf[:, None], szf[:, None], Wd)
    agg = _sc_agg(proj, idxg, dirf, cr)
    O = Wc.shape[1]
    btile = jnp.tile(b[None, :], (1, A))
    return center, agg, btile, O


def kernel(vertices, W0, b0, Wc1, Wn1, Wd1, b1, Wc2, Wn2, Wd2, b2, Wc3, Wn3, Wd3, b3, Wc4, Wn4, Wd4, b4, g1, be1, g2, be2, g3, be3):
    B, V, _ = vertices.shape
    N0 = B * V

    idx0, sx0, sy0, sz0 = _knn_pallas(vertices)
    gid0 = (idx0 + (jnp.arange(B, dtype=jnp.int32) * V)[:, None, None]).reshape(-1)
    sxf0, syf0, szf0 = _flat_sup(sx0), _flat_sup(sy0), _flat_sup(sz0)

    W0r = W0.reshape(3, A * 32)
    b0r = b0.reshape(1, A * 32)
    F0, fea0f = _surface_pallas(sxf0[:, None], syf0[:, None], szf0[:, None],
                                W0r, b0r, N0)

    c1, a1, bt1, O1 = _equi_layer(F0, gid0, sxf0, syf0, szf0,
                                  Wc1, Wn1, Wd1, b1, 512, 4)
    pre1, ps1, pq1 = _e1_pallas(c1, a1, bt1)
    F1, fea1f = _e2_pallas(pre1, ps1, pq1, g1[None, :], be1[None, :])

    V1 = V // 4
    selg = (jnp.arange(B, dtype=jnp.int32) * V)[:, None] + \
        (jnp.arange(V1, dtype=jnp.int32) * 4)[None, :]
    nbrg = gid0.reshape(B, V, K)[:, ::4, :4]
    idxp1 = jnp.concatenate(
        [selg[:, :, None], nbrg, jnp.tile(selg[:, :, None], (1, 1, 3))],
        axis=2).reshape(-1)
    Fp1 = _sc_pool(F1, idxp1, 4)
    v1 = vertices[:, ::4, :]

    idx1, sx1, sy1, sz1 = _knn_pallas(v1)
    gid1 = (idx1 + (jnp.arange(B, dtype=jnp.int32) * V1)[:, None, None]).reshape(-1)
    sxf1, syf1, szf1 = _flat_sup(sx1), _flat_sup(sy1), _flat_sup(sz1)

    c2, a2, bt2, O2 = _equi_layer(Fp1, gid1, sxf1, syf1, szf1,
                                  Wc2, Wn2, Wd2, b2, 512, 4)
    pre2, ps2, pq2 = _e1_pallas(c2, a2, bt2)
    F2, fea2f = _e2_pallas(pre2, ps2, pq2, g2[None, :], be2[None, :])

    c3, a3, bt3, O3 = _equi_layer(F2, gid1, sxf1, syf1, szf1,
                                  Wc3, Wn3, Wd3, b3, 512, 2)
    pre3, ps3, pq3 = _e1_pallas(c3, a3, bt3)
    F3, fea3f = _e2_pallas(pre3, ps3, pq3, g3[None, :], be3[None, :])

    V2 = V1 // 4
    selg2 = (jnp.arange(B, dtype=jnp.int32) * V1)[:, None] + \
        (jnp.arange(V2, dtype=jnp.int32) * 4)[None, :]
    nbrg2 = gid1.reshape(B, V1, K)[:, ::4, :4]
    idxp2 = jnp.concatenate(
        [selg2[:, :, None], nbrg2, jnp.tile(selg2[:, :, None], (1, 1, 3))],
        axis=2).reshape(-1)
    Fp2 = _sc_pool(F3, idxp2, 4)
    v2 = v1[:, ::4, :]

    idx2, sx2, sy2, sz2 = _knn_pallas(v2)
    gid2 = (idx2 + (jnp.arange(B, dtype=jnp.int32) * V2)[:, None, None]).reshape(-1)
    sxf2, syf2, szf2 = _flat_sup(sx2), _flat_sup(sy2), _flat_sup(sz2)

    c4, a4, bt4, O4 = _equi_layer(Fp2, gid2, sxf2, syf2, szf2,
                                  Wc4, Wn4, Wd4, b4, 512, 2)
    fea4f = _e4_pallas(c4, a4, bt4, 512)

    fea0 = fea0f.reshape(B, V, 32)
    fea1 = fea1f.reshape(B, V, O1)
    fea2 = fea2f.reshape(B, V1, O2)
    fea3 = fea3f.reshape(B, V1, O3)
    fea4 = fea4f.reshape(B, V2, O4)
    return (vertices, fea0, fea1, v1, fea2, fea3, v2, fea4)
